# Initial kernel scaffold; baseline (speedup 1.0000x reference)
import jax
import jax.numpy as jnp
from jax.experimental import pallas as pl


def kernel(x, embeddings):
    raise NotImplementedError("write your pallas kernel here")


# speedup vs baseline: 1.0989x; 1.0989x over previous
import functools

import jax
import jax.numpy as jnp
from jax import lax
from jax.experimental import pallas as pl
from jax.experimental.pallas import tpu as pltpu
from jax.experimental.pallas import tpu_sc as plsc

NUM_EMB = 1000000
D = 32
B = 16384 * 50
NC, NS = 2, 16
NW = NC * NS
BPW = B // NW
CHUNK = 1280
NCHUNK = BPW // CHUNK

_mesh = plsc.VectorSubcoreMesh(core_axis_name="c", subcore_axis_name="s")


@functools.partial(
    pl.kernel,
    mesh=_mesh,
    out_type=jax.ShapeDtypeStruct((B, D), jnp.float32),
    scratch_types=[
        pltpu.VMEM((CHUNK,), jnp.int32),
        pltpu.VMEM((CHUNK, D), jnp.float32),
        pltpu.SemaphoreType.DMA,
    ],
    compiler_params=pltpu.CompilerParams(use_tc_tiling_on_sc=False),
)
def _gather_kernel(idx_hbm, table_hbm, out_hbm, idx_v, rows_v, sem):
    wid = lax.axis_index("s") * NC + lax.axis_index("c")
    base = wid * BPW

    def body(g, carry):
        off = base + g * CHUNK
        pltpu.sync_copy(idx_hbm.at[pl.ds(off, CHUNK)], idx_v)
        pltpu.async_copy(table_hbm.at[idx_v], rows_v, sem).wait()
        pltpu.sync_copy(rows_v, out_hbm.at[pl.ds(off, CHUNK)])
        return carry

    lax.fori_loop(0, NCHUNK, body, 0)


def kernel(x, embeddings):
    flat_idx = x.reshape(-1)
    out = _gather_kernel(flat_idx, embeddings)
    return out.reshape(x.shape[0], x.shape[1], D)


# speedup vs baseline: 1.1088x; 1.0090x over previous
import functools

import jax
import jax.numpy as jnp
from jax import lax
from jax.experimental import pallas as pl
from jax.experimental.pallas import tpu as pltpu
from jax.experimental.pallas import tpu_sc as plsc

D = 32
B = 16384 * 50
NC, NS = 2, 16
NW = NC * NS
BPW = B // NW
CHUNK = 1600
NCHUNK = BPW // CHUNK
NBUF = 2

_mesh = plsc.VectorSubcoreMesh(core_axis_name="c", subcore_axis_name="s")


@functools.partial(
    pl.kernel,
    mesh=_mesh,
    out_type=jax.ShapeDtypeStruct((B, D), jnp.float32),
    scratch_types=[
        pltpu.VMEM((NBUF, CHUNK), jnp.int32),
        pltpu.VMEM((NBUF, CHUNK, D), jnp.float32),
        pltpu.SemaphoreType.DMA,
        pltpu.SemaphoreType.DMA,
        pltpu.SemaphoreType.DMA,
        pltpu.SemaphoreType.DMA,
    ],
    compiler_params=pltpu.CompilerParams(use_tc_tiling_on_sc=False),
)
def _gather_kernel(idx_hbm, table_hbm, out_hbm, idx_v, rows_v, sg0, sg1, sw0, sw1):
    wid = lax.axis_index("s") * NC + lax.axis_index("c")
    base = wid * BPW
    sem_g = (sg0, sg1)
    sem_w = (sw0, sw1)
    gathers = [None] * NCHUNK
    writebacks = [None] * NCHUNK

    for g in range(NCHUNK):
        b = g % NBUF
        off = base + g * CHUNK
        if g >= NBUF:
            writebacks[g - NBUF].wait()
        pltpu.sync_copy(idx_hbm.at[pl.ds(off, CHUNK)], idx_v.at[b])
        gathers[g] = pltpu.async_copy(table_hbm.at[idx_v.at[b]], rows_v.at[b], sem_g[b])
        if g >= 1:
            gathers[g - 1].wait()
            writebacks[g - 1] = pltpu.async_copy(
                rows_v.at[1 - b], out_hbm.at[pl.ds(off - CHUNK, CHUNK)], sem_w[1 - b]
            )

    last = NCHUNK - 1
    gathers[last].wait()
    writebacks[last] = pltpu.async_copy(
        rows_v.at[last % NBUF], out_hbm.at[pl.ds(base + last * CHUNK, CHUNK)], sem_w[last % NBUF]
    )
    writebacks[last - 1].wait()
    writebacks[last].wait()


def kernel(x, embeddings):
    flat_idx = x.reshape(-1)
    out = _gather_kernel(flat_idx, embeddings)
    return out.reshape(x.shape[0], x.shape[1], D)
